# baseline (speedup 1.0000x reference)
import jax
import jax.numpy as jnp
from jax.experimental import pallas as pl

_N_Q = 8
_K = 1024
_D = 128
_TILE = 1200


def _rvq_body(x_ref, cb_ref, out_ref):
    r = x_ref[...]
    iota = jax.lax.broadcasted_iota(jnp.int32, (_TILE, _K), 1)
    for q in range(_N_Q):
        cb = cb_ref[q]
        s = jax.lax.dot_general(
            r, cb, (((1,), (1,)), ((), ())),
            preferred_element_type=jnp.float32,
            precision=jax.lax.Precision.DEFAULT)
        cbn = jnp.sum(cb * cb, axis=1)
        rss = jnp.sum(r * r, axis=1, keepdims=True)
        d = (rss - 2.0 * s) + cbn[None, :]
        m = jnp.min(d, axis=1, keepdims=True)
        idx = jnp.min(jnp.where(d == m, iota, _K), axis=1, keepdims=True)
        onehot = (iota == idx).astype(jnp.float32)
        quant = jax.lax.dot_general(
            onehot, cb, (((1,), (0,)), ((), ())),
            preferred_element_type=jnp.float32,
            precision=jax.lax.Precision.HIGHEST)
        r = r - quant
        out_ref[:, q:q + 1] = idx


def kernel(wav_features, codebooks):
    B, T, D = wav_features.shape
    n = B * T
    x = wav_features.reshape(n, D)
    grid = n // _TILE
    out = pl.pallas_call(
        _rvq_body,
        grid=(grid,),
        in_specs=[
            pl.BlockSpec((_TILE, _D), lambda i: (i, 0)),
            pl.BlockSpec((_N_Q, _K, _D), lambda i: (0, 0, 0)),
        ],
        out_specs=pl.BlockSpec((_TILE, _N_Q), lambda i: (i, 0)),
        out_shape=jax.ShapeDtypeStruct((n, _N_Q), jnp.int32),
    )(x, codebooks)
    return out.T.reshape(_N_Q, B, T)


# speedup vs baseline: 3.2957x; 3.2957x over previous
import jax
import jax.numpy as jnp
from jax.experimental import pallas as pl

_N_Q = 8
_K = 1024
_D = 128
_TILE = 1200


def _rvq_body(x_ref, cb_ref, cbp_ref, out_ref):
    r = x_ref[...]
    iota = jax.lax.broadcasted_iota(jnp.int32, (_TILE, _K), 1)
    codes = []
    for q in range(_N_Q):
        cb = cb_ref[q]
        s = jax.lax.dot_general(
            r, cb, (((1,), (1,)), ((), ())),
            preferred_element_type=jnp.float32,
            precision=jax.lax.Precision.DEFAULT)
        cbn = jnp.sum(cb * cb, axis=1)
        rss = jnp.sum(r * r, axis=1, keepdims=True)
        d = (rss - 2.0 * s) + cbn[None, :]
        m = jnp.min(d, axis=1, keepdims=True)
        t = jnp.where(d == m, iota, _K)
        idx = jnp.min(t, axis=1, keepdims=True)
        onehot = (iota == idx).astype(jnp.bfloat16)
        g = jax.lax.dot_general(
            onehot, cbp_ref[q], (((1,), (0,)), ((), ())),
            preferred_element_type=jnp.float32,
            precision=jax.lax.Precision.DEFAULT)
        r = r - ((g[:, :_D] + g[:, _D:2 * _D]) + g[:, 2 * _D:])
        out_ref[:, q:q + 1] = idx


def kernel(wav_features, codebooks):
    B, T, D = wav_features.shape
    n = B * T
    x = wav_features.reshape(n, D)
    bits = jax.lax.bitcast_convert_type(codebooks, jnp.uint32)
    hi = jax.lax.bitcast_convert_type(bits & jnp.uint32(0xFFFF0000),
                                      jnp.float32)
    r1 = codebooks - hi
    r1b = jax.lax.bitcast_convert_type(r1, jnp.uint32)
    mid = jax.lax.bitcast_convert_type(r1b & jnp.uint32(0xFFFF0000),
                                       jnp.float32)
    lo = r1 - mid
    cbp = jnp.concatenate(
        [hi.astype(jnp.bfloat16), mid.astype(jnp.bfloat16),
         lo.astype(jnp.bfloat16)], axis=-1)
    grid = n // _TILE
    out = pl.pallas_call(
        _rvq_body,
        grid=(grid,),
        in_specs=[
            pl.BlockSpec((_TILE, _D), lambda i: (i, 0)),
            pl.BlockSpec((_N_Q, _K, _D), lambda i: (0, 0, 0)),
            pl.BlockSpec((_N_Q, _K, 3 * _D), lambda i: (0, 0, 0)),
        ],
        out_specs=pl.BlockSpec((_TILE, _N_Q), lambda i: (i, 0)),
        out_shape=jax.ShapeDtypeStruct((n, _N_Q), jnp.int32),
    )(x, codebooks, cbp)
    return out.T.reshape(_N_Q, B, T)
